# baseline (speedup 1.0000x reference)
import functools

import jax
import jax.numpy as jnp
from jax import lax
from jax.experimental import pallas as pl
from jax.experimental.pallas import tpu as pltpu
from jax.experimental.pallas import tpu_sc as plsc

N = 10000
E = 320000
D = 128
G = 64

L = 16
NGRP = D // L
K = 40


def _pre_body(h_ref, wm1_ref, wm2_ref, bmsg_ref, a_ref, b_ref):
    h = h_ref[...]
    a_ref[...] = jnp.dot(h, wm1_ref[...], preferred_element_type=jnp.float32)
    b_ref[...] = (jnp.dot(h, wm2_ref[...], preferred_element_type=jnp.float32)
                  + bmsg_ref[...])


def _tc_pre(h, wm1, wm2, bmsg):
    return pl.pallas_call(
        _pre_body,
        out_shape=(jax.ShapeDtypeStruct((N, D), jnp.float32),
                   jax.ShapeDtypeStruct((N, D), jnp.float32)),
    )(h, wm1, wm2, bmsg)


def _post_body(h_ref, p0_ref, p1_ref, wres_ref, wu1_ref, wu2_ref, bupd_ref,
               out_ref):
    h = h_ref[...]
    aggr = p0_ref[...] + p1_ref[...]
    u = (jnp.dot(h, wu1_ref[...], preferred_element_type=jnp.float32)
         + jnp.dot(aggr, wu2_ref[...], preferred_element_type=jnp.float32)
         + bupd_ref[...])
    out_ref[...] = (jnp.dot(h, wres_ref[...], preferred_element_type=jnp.float32)
                    + jnp.maximum(u, 0.0))


def _tc_post(h, p0, p1, wres, wu1, wu2, bupd):
    return pl.pallas_call(
        _post_body,
        out_shape=jax.ShapeDtypeStruct((N, D), jnp.float32),
    )(h, p0, p1, wres, wu1, wu2, bupd)


def _readout_body(h_ref, batch_ref, wp1_ref, bp1_ref, wp2_ref, bp2_ref,
                  out_ref):
    h = h_ref[...]
    b = batch_ref[0:1, :]
    gids = lax.broadcasted_iota(jnp.int32, (G, N), 0)
    onehot = (gids == b).astype(jnp.float32)
    g = jnp.dot(onehot, h, preferred_element_type=jnp.float32)
    t = jnp.maximum(
        jnp.dot(g, wp1_ref[...], preferred_element_type=jnp.float32)
        + bp1_ref[...], 0.0)
    out_ref[...] = (jnp.dot(t, wp2_ref[...], preferred_element_type=jnp.float32)
                    + bp2_ref[...])


def _tc_readout(h, batch8, wp1, bp1, wp2pad, bp2b):
    return pl.pallas_call(
        _readout_body,
        out_shape=jax.ShapeDtypeStruct((G, D), jnp.float32),
    )(h, batch8, wp1, bp1, wp2pad, bp2b)


def _sc_edge_body(a_hbm, b_hbm, wv_hbm, row_hbm, col_hbm, ea_hbm, out_hbm,
                  idxr0, idxc0, ea0, a0, b0, m0,
                  idxr1, idxc1, ea1, a1, b1, m1,
                  wv_v, aggr_sh, sem_a0, sem_b0, sem_a1, sem_b1):
    nc = 2
    ns = 16
    c = lax.axis_index("c")
    s = lax.axis_index("s")
    w = s * nc + c
    epw = E // (nc * ns)
    zrows = K
    nchunk = N // zrows

    def _zero_row(i, _):
        for gi in range(NGRP):
            m0[i, pl.ds(gi * L, L)] = jnp.zeros((L,), jnp.float32)
        return 0
    lax.fori_loop(0, zrows, _zero_row, 0)
    for t in range(pl.cdiv(nchunk, ns)):
        cid = t * ns + s
        @pl.when(cid < nchunk)
        def _():
            pltpu.sync_copy(m0, aggr_sh.at[pl.ds(cid * zrows, zrows)])
    plsc.subcore_barrier()

    pltpu.sync_copy(wv_hbm, wv_v)
    wvs = [wv_v[pl.ds(gi * L, L)] for gi in range(NGRP)]

    base = w * epw
    nblk = epw // K
    bufs = ((idxr0, idxc0, ea0, a0, b0, m0, sem_a0, sem_b0),
            (idxr1, idxc1, ea1, a1, b1, m1, sem_a1, sem_b1))

    def _fire(blk, buf):
        idxr, idxc, eav, av, bv, _, sa, sb = buf
        off = base + blk * K
        pltpu.sync_copy(row_hbm.at[pl.ds(off, K)], idxr)
        pltpu.sync_copy(col_hbm.at[pl.ds(off, K)], idxc)
        pltpu.sync_copy(ea_hbm.at[pl.ds(off * L, K * L)], eav)
        pltpu.async_copy(a_hbm.at[idxr], av, sa)
        pltpu.async_copy(b_hbm.at[idxc], bv, sb)

    def _drain(buf):
        idxr, idxc, _, av, bv, _, sa, sb = buf
        pltpu.make_async_copy(a_hbm.at[idxr], av, sa).wait()
        pltpu.make_async_copy(b_hbm.at[idxc], bv, sb).wait()

    def _compute_scatter(buf):
        idxr, idxc, eav, av, bv, mv, _, _ = buf

        @plsc.parallel_loop(0, K, unroll=4)
        def _edge(j):
            ea_s = eav[pl.ds(j * L, L)]
            for gi in range(NGRP):
                sl = pl.ds(gi * L, L)
                m = av[j, sl] + bv[j, sl] + ea_s * wvs[gi]
                mv[j, sl] = jnp.maximum(m, 0.0)
        pltpu.sync_copy(mv, aggr_sh.at[idxc], add=True)

    _fire(0, bufs[0])
    _fire(1, bufs[1])

    def _pair(i, _):
        _drain(bufs[0])
        _compute_scatter(bufs[0])
        _fire(2 * i + 2, bufs[0])
        _drain(bufs[1])
        _compute_scatter(bufs[1])
        _fire(2 * i + 3, bufs[1])
        return 0

    lax.fori_loop(0, nblk // 2 - 1, _pair, 0)
    for buf in bufs:
        _drain(buf)
        _compute_scatter(buf)
    plsc.subcore_barrier()

    for t in range(pl.cdiv(nchunk, ns)):
        cid = t * ns + s
        @pl.when(cid < nchunk)
        def _():
            r0 = cid * zrows
            pltpu.sync_copy(aggr_sh.at[pl.ds(r0, zrows)], m0)
            pltpu.sync_copy(m0, out_hbm.at[c, pl.ds(r0, zrows)])


def _sc_edge(a, b, wv, row, col, ea):
    mesh = plsc.VectorSubcoreMesh(core_axis_name="c", subcore_axis_name="s")
    fn = functools.partial(
        pl.kernel,
        out_type=jax.ShapeDtypeStruct((2, N, D), jnp.float32),
        mesh=mesh,
        scratch_types=(
            [pltpu.VMEM((K,), jnp.int32),
             pltpu.VMEM((K,), jnp.int32),
             pltpu.VMEM((K * L,), jnp.float32),
             pltpu.VMEM((K, D), jnp.float32),
             pltpu.VMEM((K, D), jnp.float32),
             pltpu.VMEM((K, D), jnp.float32)] * 2
            + [
                pltpu.VMEM((D,), jnp.float32),
                pltpu.VMEM_SHARED((N, D), jnp.float32),
                pltpu.SemaphoreType.DMA,
                pltpu.SemaphoreType.DMA,
                pltpu.SemaphoreType.DMA,
                pltpu.SemaphoreType.DMA,
            ]),
    )(_sc_edge_body)
    return fn(a, b, wv, row, col, ea)


def kernel(x, edge_attr, edge_index, batch,
           Wres0, Wmsg0, bmsg0, Wupd0, bupd0,
           Wres1, Wmsg1, bmsg1, Wupd1, bupd1,
           Wp1, bp1, Wp2, bp2):
    row = edge_index[0]
    col = edge_index[1]
    ea = jnp.broadcast_to(edge_attr.reshape(E, 1), (E, L)).reshape(E * L)

    h = x
    for (Wres, Wmsg, bmsg, Wupd, bupd) in (
            (Wres0, Wmsg0, bmsg0, Wupd0, bupd0),
            (Wres1, Wmsg1, bmsg1, Wupd1, bupd1)):
        wm1 = Wmsg[:D]
        wm2 = Wmsg[D:2 * D]
        wv = Wmsg[2 * D]
        wu1 = Wupd[:D]
        wu2 = Wupd[D:]
        a, b = _tc_pre(h, wm1, wm2, bmsg.reshape(1, D))
        parts = _sc_edge(a, b, wv, row, col, ea)
        h = _tc_post(h, parts[0], parts[1], Wres, wu1, wu2, bupd.reshape(1, D))

    batch8 = jnp.broadcast_to(batch.reshape(1, N), (8, N))
    wp2pad = jnp.pad(Wp2, ((0, 0), (0, D - 1)))
    bp2b = jnp.broadcast_to(bp2.reshape(1, 1), (1, D))
    out = _tc_readout(h, batch8, Wp1, bp1.reshape(1, D), wp2pad, bp2b)
    return out[:, :1]


# speedup vs baseline: 1.2470x; 1.2470x over previous
import functools

import jax
import jax.numpy as jnp
from jax import lax
from jax.experimental import pallas as pl
from jax.experimental.pallas import tpu as pltpu
from jax.experimental.pallas import tpu_sc as plsc

N = 10000
E = 320000
D = 128
G = 64

L = 16
NGRP = D // L
K = 40


def _pre_body(h_ref, wm1_ref, wm2_ref, bmsg_ref, a_ref, b_ref):
    h = h_ref[...]
    a_ref[...] = jnp.dot(h, wm1_ref[...], preferred_element_type=jnp.float32)
    b_ref[...] = (jnp.dot(h, wm2_ref[...], preferred_element_type=jnp.float32)
                  + bmsg_ref[...])


def _tc_pre(h, wm1, wm2, bmsg):
    return pl.pallas_call(
        _pre_body,
        out_shape=(jax.ShapeDtypeStruct((N, D), jnp.float32),
                   jax.ShapeDtypeStruct((N, D), jnp.float32)),
    )(h, wm1, wm2, bmsg)


def _post_body(h_ref, p0_ref, p1_ref, wres_ref, wu1_ref, wu2_ref, bupd_ref,
               out_ref):
    h = h_ref[...]
    aggr = p0_ref[...] + p1_ref[...]
    u = (jnp.dot(h, wu1_ref[...], preferred_element_type=jnp.float32)
         + jnp.dot(aggr, wu2_ref[...], preferred_element_type=jnp.float32)
         + bupd_ref[...])
    out_ref[...] = (jnp.dot(h, wres_ref[...], preferred_element_type=jnp.float32)
                    + jnp.maximum(u, 0.0))


def _tc_post(h, p0, p1, wres, wu1, wu2, bupd):
    return pl.pallas_call(
        _post_body,
        out_shape=jax.ShapeDtypeStruct((N, D), jnp.float32),
    )(h, p0, p1, wres, wu1, wu2, bupd)


def _readout_body(h_ref, batch_ref, wp1_ref, bp1_ref, wp2_ref, bp2_ref,
                  out_ref):
    h = h_ref[...]
    b = batch_ref[0:1, :]
    gids = lax.broadcasted_iota(jnp.int32, (G, N), 0)
    onehot = (gids == b).astype(jnp.float32)
    g = jnp.dot(onehot, h, preferred_element_type=jnp.float32)
    t = jnp.maximum(
        jnp.dot(g, wp1_ref[...], preferred_element_type=jnp.float32)
        + bp1_ref[...], 0.0)
    out_ref[...] = (jnp.dot(t, wp2_ref[...], preferred_element_type=jnp.float32)
                    + bp2_ref[...])


def _tc_readout(h, batch8, wp1, bp1, wp2pad, bp2b):
    return pl.pallas_call(
        _readout_body,
        out_shape=jax.ShapeDtypeStruct((G, D), jnp.float32),
    )(h, batch8, wp1, bp1, wp2pad, bp2b)


def _sc_edge_body(a_hbm, b_hbm, wv_hbm, col_hbm, row_hbm, ea_hbm, out_hbm,
                  col0, row0, ea0, a0, b0, m0,
                  col1, row1, ea1, a1, b1, m1,
                  col2, row2, ea2, a2, b2, m2,
                  wv_v, aggr_sh,
                  si0, sg0, ss0, si1, sg1, ss1, si2, sg2, ss2):
    nc = 2
    ns = 16
    c = lax.axis_index("c")
    s = lax.axis_index("s")
    w = s * nc + c
    epw = E // (nc * ns)
    zrows = K
    nchunk = N // zrows
    nblk = epw // K
    base_e = w * epw
    bufs = ((col0, row0, ea0, a0, b0, m0, si0, sg0, ss0),
            (col1, row1, ea1, a1, b1, m1, si1, sg1, ss1),
            (col2, row2, ea2, a2, b2, m2, si2, sg2, ss2))

    def _zero_row(i, _):
        for gi in range(NGRP):
            m0[i, pl.ds(gi * L, L)] = jnp.zeros((L,), jnp.float32)
        return 0
    lax.fori_loop(0, zrows, _zero_row, 0)
    for t in range(pl.cdiv(nchunk, ns)):
        cid = t * ns + s
        @pl.when(cid < nchunk)
        def _():
            pltpu.sync_copy(m0, aggr_sh.at[pl.ds(cid * zrows, zrows)])
    plsc.subcore_barrier()

    pltpu.sync_copy(wv_hbm, wv_v)
    wvs = [wv_v[pl.ds(gi * L, L)] for gi in range(NGRP)]

    def _fire_idx(blk, buf):
        colv, rowv, eav, _, _, _, si, _, _ = buf
        off = base_e + blk * K
        pltpu.async_copy(col_hbm.at[pl.ds(off, K)], colv, si)
        pltpu.async_copy(row_hbm.at[pl.ds(off, K)], rowv, si)
        pltpu.async_copy(ea_hbm.at[pl.ds(off * L, K * L)], eav, si)

    def _drain_idx(buf):
        colv, rowv, eav, _, _, _, si, _, _ = buf
        pltpu.make_async_copy(col_hbm.at[pl.ds(0, K)], colv, si).wait()
        pltpu.make_async_copy(row_hbm.at[pl.ds(0, K)], rowv, si).wait()
        pltpu.make_async_copy(ea_hbm.at[pl.ds(0, K * L)], eav, si).wait()

    def _fire_gathers(buf):
        colv, rowv, _, av, bv, _, _, sg, _ = buf
        pltpu.async_copy(a_hbm.at[rowv], av, sg)
        pltpu.async_copy(b_hbm.at[colv], bv, sg)

    def _drain_gathers(buf):
        colv, rowv, _, av, bv, _, _, sg, _ = buf
        pltpu.make_async_copy(a_hbm.at[rowv], av, sg).wait()
        pltpu.make_async_copy(b_hbm.at[colv], bv, sg).wait()

    def _compute(buf):
        _, _, eav, av, bv, mv, _, _, _ = buf

        @plsc.parallel_loop(0, K, unroll=4)
        def _edge(j):
            ea_s = eav[pl.ds(j * L, L)]
            for gi in range(NGRP):
                sl = pl.ds(gi * L, L)
                m = av[j, sl] + bv[j, sl] + ea_s * wvs[gi]
                mv[j, sl] = jnp.maximum(m, 0.0)

    def _fire_scatter(buf):
        colv, _, _, _, _, mv, _, _, ss = buf
        pltpu.async_copy(mv, aggr_sh.at[colv], ss, add=True)

    def _drain_scatter(buf):
        colv, _, _, _, _, mv, _, _, ss = buf
        pltpu.make_async_copy(mv, aggr_sh.at[colv], ss).wait()

    _fire_idx(0, bufs[0])
    _fire_idx(1, bufs[1])
    _drain_idx(bufs[0])
    _fire_gathers(bufs[0])

    def _step(t, p):
        buf_p = bufs[p]
        buf_q = bufs[(p + 1) % 3]
        buf_r = bufs[(p + 2) % 3]

        @pl.when(t < nblk)
        def _():
            _drain_gathers(buf_p)
            _compute(buf_p)
            _fire_scatter(buf_p)

        @pl.when(t + 1 < nblk)
        def _():
            _drain_idx(buf_q)
            _fire_gathers(buf_q)

        @pl.when(jnp.logical_and(t >= 1, t <= nblk))
        def _():
            _drain_scatter(buf_r)

        @pl.when(t + 2 < nblk)
        def _():
            _fire_idx(t + 2, buf_r)

    def _triple(i, _):
        for ps in range(3):
            _step(3 * i + ps, ps)
        return 0

    lax.fori_loop(0, pl.cdiv(nblk + 1, 3), _triple, 0)
    plsc.subcore_barrier()

    for t in range(pl.cdiv(nchunk, ns)):
        cid = t * ns + s
        @pl.when(cid < nchunk)
        def _():
            r0 = cid * zrows
            pltpu.sync_copy(aggr_sh.at[pl.ds(r0, zrows)], m0)
            pltpu.sync_copy(m0, out_hbm.at[c, pl.ds(r0, zrows)])


def _sc_edge(a, b, wv, col, row, ea):
    mesh = plsc.VectorSubcoreMesh(core_axis_name="c", subcore_axis_name="s")
    fn = functools.partial(
        pl.kernel,
        out_type=jax.ShapeDtypeStruct((2, N, D), jnp.float32),
        mesh=mesh,
        scratch_types=(
            [pltpu.VMEM((K,), jnp.int32),
             pltpu.VMEM((K,), jnp.int32),
             pltpu.VMEM((K * L,), jnp.float32),
             pltpu.VMEM((K, D), jnp.float32),
             pltpu.VMEM((K, D), jnp.float32),
             pltpu.VMEM((K, D), jnp.float32)] * 3
            + [
                pltpu.VMEM((D,), jnp.float32),
                pltpu.VMEM_SHARED((N, D), jnp.float32),
            ]
            + [pltpu.SemaphoreType.DMA] * 9),
    )(_sc_edge_body)
    return fn(a, b, wv, col, row, ea)


def kernel(x, edge_attr, edge_index, batch,
           Wres0, Wmsg0, bmsg0, Wupd0, bupd0,
           Wres1, Wmsg1, bmsg1, Wupd1, bupd1,
           Wp1, bp1, Wp2, bp2):
    row = edge_index[0]
    col = edge_index[1]
    ea = jnp.broadcast_to(edge_attr.reshape(E, 1), (E, L)).reshape(E * L)

    h = x
    for (Wres, Wmsg, bmsg, Wupd, bupd) in (
            (Wres0, Wmsg0, bmsg0, Wupd0, bupd0),
            (Wres1, Wmsg1, bmsg1, Wupd1, bupd1)):
        wm1 = Wmsg[:D]
        wm2 = Wmsg[D:2 * D]
        wv = Wmsg[2 * D]
        wu1 = Wupd[:D]
        wu2 = Wupd[D:]
        a, b = _tc_pre(h, wm1, wm2, bmsg.reshape(1, D))
        parts = _sc_edge(a, b, wv, col, row, ea)
        h = _tc_post(h, parts[0], parts[1], Wres, wu1, wu2, bupd.reshape(1, D))

    batch8 = jnp.broadcast_to(batch.reshape(1, N), (8, N))
    wp2pad = jnp.pad(Wp2, ((0, 0), (0, D - 1)))
    bp2b = jnp.broadcast_to(bp2.reshape(1, 1), (1, D))
    out = _tc_readout(h, batch8, Wp1, bp1.reshape(1, D), wp2pad, bp2b)
    return out[:, :1]


# speedup vs baseline: 1.2565x; 1.0077x over previous
import functools

import jax
import jax.numpy as jnp
from jax import lax
from jax.experimental import pallas as pl
from jax.experimental.pallas import tpu as pltpu
from jax.experimental.pallas import tpu_sc as plsc

N = 10000
E = 320000
D = 128
G = 64

L = 16
NGRP = D // L
K = 40


def _pre_body(h_ref, wm1_ref, wm2_ref, bmsg_ref, a_ref, b_ref):
    h = h_ref[...]
    a_ref[...] = jnp.dot(h, wm1_ref[...], preferred_element_type=jnp.float32)
    b_ref[...] = (jnp.dot(h, wm2_ref[...], preferred_element_type=jnp.float32)
                  + bmsg_ref[...])


def _tc_pre(h, wm1, wm2, bmsg):
    return pl.pallas_call(
        _pre_body,
        out_shape=(jax.ShapeDtypeStruct((N, D), jnp.float32),
                   jax.ShapeDtypeStruct((N, D), jnp.float32)),
    )(h, wm1, wm2, bmsg)


def _node_update(h, p0_ref, p1_ref, wres_ref, wu1_ref, wu2_ref, bupd_ref):
    aggr = p0_ref[...] + p1_ref[...]
    u = (jnp.dot(h, wu1_ref[...], preferred_element_type=jnp.float32)
         + jnp.dot(aggr, wu2_ref[...], preferred_element_type=jnp.float32)
         + bupd_ref[...])
    return (jnp.dot(h, wres_ref[...], preferred_element_type=jnp.float32)
            + jnp.maximum(u, 0.0))


def _post_pre_body(h_ref, p0_ref, p1_ref, wres_ref, wu1_ref, wu2_ref,
                   bupd_ref, wm1_ref, wm2_ref, bmsg_ref,
                   h_out, a_out, b_out):
    hn = _node_update(h_ref[...], p0_ref, p1_ref, wres_ref, wu1_ref, wu2_ref,
                      bupd_ref)
    h_out[...] = hn
    a_out[...] = jnp.dot(hn, wm1_ref[...], preferred_element_type=jnp.float32)
    b_out[...] = (jnp.dot(hn, wm2_ref[...], preferred_element_type=jnp.float32)
                  + bmsg_ref[...])


def _tc_post_pre(h, p0, p1, wres, wu1, wu2, bupd, wm1, wm2, bmsg):
    return pl.pallas_call(
        _post_pre_body,
        out_shape=(jax.ShapeDtypeStruct((N, D), jnp.float32),
                   jax.ShapeDtypeStruct((N, D), jnp.float32),
                   jax.ShapeDtypeStruct((N, D), jnp.float32)),
    )(h, p0, p1, wres, wu1, wu2, bupd, wm1, wm2, bmsg)


def _post_readout_body(h_ref, p0_ref, p1_ref, wres_ref, wu1_ref, wu2_ref,
                       bupd_ref, batch_ref, wp1_ref, bp1_ref, wp2_ref,
                       bp2_ref, out_ref):
    hn = _node_update(h_ref[...], p0_ref, p1_ref, wres_ref, wu1_ref, wu2_ref,
                      bupd_ref)
    b = batch_ref[0:1, :]
    gids = lax.broadcasted_iota(jnp.int32, (G, N), 0)
    onehot = (gids == b).astype(jnp.float32)
    g = jnp.dot(onehot, hn, preferred_element_type=jnp.float32)
    t = jnp.maximum(
        jnp.dot(g, wp1_ref[...], preferred_element_type=jnp.float32)
        + bp1_ref[...], 0.0)
    out_ref[...] = (jnp.dot(t, wp2_ref[...], preferred_element_type=jnp.float32)
                    + bp2_ref[...])


def _tc_post_readout(h, p0, p1, wres, wu1, wu2, bupd, batch8, wp1, bp1,
                     wp2pad, bp2b):
    return pl.pallas_call(
        _post_readout_body,
        out_shape=jax.ShapeDtypeStruct((G, D), jnp.float32),
    )(h, p0, p1, wres, wu1, wu2, bupd, batch8, wp1, bp1, wp2pad, bp2b)


def _sc_edge_body(a_hbm, b_hbm, wv_hbm, col_hbm, row_hbm, ea_hbm, out_hbm,
                  col0, row0, ea0, a0, b0, m0,
                  col1, row1, ea1, a1, b1, m1,
                  col2, row2, ea2, a2, b2, m2,
                  wv_v, aggr_sh,
                  si0, sg0, ss0, si1, sg1, ss1, si2, sg2, ss2):
    nc = 2
    ns = 16
    c = lax.axis_index("c")
    s = lax.axis_index("s")
    w = s * nc + c
    epw = E // (nc * ns)
    zrows = K
    nchunk = N // zrows
    nblk = epw // K
    base_e = w * epw
    bufs = ((col0, row0, ea0, a0, b0, m0, si0, sg0, ss0),
            (col1, row1, ea1, a1, b1, m1, si1, sg1, ss1),
            (col2, row2, ea2, a2, b2, m2, si2, sg2, ss2))

    def _zero_row(i, _):
        for gi in range(NGRP):
            m0[i, pl.ds(gi * L, L)] = jnp.zeros((L,), jnp.float32)
        return 0
    lax.fori_loop(0, zrows, _zero_row, 0)
    for t in range(pl.cdiv(nchunk, ns)):
        cid = t * ns + s
        @pl.when(cid < nchunk)
        def _():
            pltpu.sync_copy(m0, aggr_sh.at[pl.ds(cid * zrows, zrows)])
    plsc.subcore_barrier()

    pltpu.sync_copy(wv_hbm, wv_v)
    wvs = [wv_v[pl.ds(gi * L, L)] for gi in range(NGRP)]

    def _fire_idx(blk, buf):
        colv, rowv, eav, _, _, _, si, _, _ = buf
        off = base_e + blk * K
        pltpu.async_copy(col_hbm.at[pl.ds(off, K)], colv, si)
        pltpu.async_copy(row_hbm.at[pl.ds(off, K)], rowv, si)
        pltpu.async_copy(ea_hbm.at[pl.ds(off * L, K * L)], eav, si)

    def _drain_idx(buf):
        colv, rowv, eav, _, _, _, si, _, _ = buf
        pltpu.make_async_copy(col_hbm.at[pl.ds(0, K)], colv, si).wait()
        pltpu.make_async_copy(row_hbm.at[pl.ds(0, K)], rowv, si).wait()
        pltpu.make_async_copy(ea_hbm.at[pl.ds(0, K * L)], eav, si).wait()

    def _fire_gathers(buf):
        colv, rowv, _, av, bv, _, _, sg, _ = buf
        pltpu.async_copy(a_hbm.at[rowv], av, sg)
        pltpu.async_copy(b_hbm.at[colv], bv, sg)

    def _drain_gathers(buf):
        colv, rowv, _, av, bv, _, _, sg, _ = buf
        pltpu.make_async_copy(a_hbm.at[rowv], av, sg).wait()
        pltpu.make_async_copy(b_hbm.at[colv], bv, sg).wait()

    def _compute(buf):
        _, _, eav, av, bv, mv, _, _, _ = buf

        @plsc.parallel_loop(0, K, unroll=4)
        def _edge(j):
            ea_s = eav[pl.ds(j * L, L)]
            for gi in range(NGRP):
                sl = pl.ds(gi * L, L)
                m = av[j, sl] + bv[j, sl] + ea_s * wvs[gi]
                mv[j, sl] = jnp.maximum(m, 0.0)

    def _fire_scatter(buf):
        colv, _, _, _, _, mv, _, _, ss = buf
        pltpu.async_copy(mv, aggr_sh.at[colv], ss, add=True)

    def _drain_scatter(buf):
        colv, _, _, _, _, mv, _, _, ss = buf
        pltpu.make_async_copy(mv, aggr_sh.at[colv], ss).wait()

    _fire_idx(0, bufs[0])
    _fire_idx(1, bufs[1])
    _drain_idx(bufs[0])
    _fire_gathers(bufs[0])

    def _step(t, p):
        buf_p = bufs[p]
        buf_q = bufs[(p + 1) % 3]
        buf_r = bufs[(p + 2) % 3]

        @pl.when(t < nblk)
        def _():
            _drain_gathers(buf_p)
            _compute(buf_p)
            _fire_scatter(buf_p)

        @pl.when(t + 1 < nblk)
        def _():
            _drain_idx(buf_q)
            _fire_gathers(buf_q)

        @pl.when(jnp.logical_and(t >= 1, t <= nblk))
        def _():
            _drain_scatter(buf_r)

        @pl.when(t + 2 < nblk)
        def _():
            _fire_idx(t + 2, buf_r)

    def _triple(i, _):
        for ps in range(3):
            _step(3 * i + ps, ps)
        return 0

    lax.fori_loop(0, pl.cdiv(nblk + 1, 3), _triple, 0)
    plsc.subcore_barrier()

    for t in range(pl.cdiv(nchunk, ns)):
        cid = t * ns + s
        @pl.when(cid < nchunk)
        def _():
            r0 = cid * zrows
            pltpu.sync_copy(aggr_sh.at[pl.ds(r0, zrows)], m0)
            pltpu.sync_copy(m0, out_hbm.at[c, pl.ds(r0, zrows)])


def _sc_edge(a, b, wv, col, row, ea):
    mesh = plsc.VectorSubcoreMesh(core_axis_name="c", subcore_axis_name="s")
    fn = functools.partial(
        pl.kernel,
        out_type=jax.ShapeDtypeStruct((2, N, D), jnp.float32),
        mesh=mesh,
        scratch_types=(
            [pltpu.VMEM((K,), jnp.int32),
             pltpu.VMEM((K,), jnp.int32),
             pltpu.VMEM((K * L,), jnp.float32),
             pltpu.VMEM((K, D), jnp.float32),
             pltpu.VMEM((K, D), jnp.float32),
             pltpu.VMEM((K, D), jnp.float32)] * 3
            + [
                pltpu.VMEM((D,), jnp.float32),
                pltpu.VMEM_SHARED((N, D), jnp.float32),
            ]
            + [pltpu.SemaphoreType.DMA] * 9),
    )(_sc_edge_body)
    return fn(a, b, wv, col, row, ea)


def kernel(x, edge_attr, edge_index, batch,
           Wres0, Wmsg0, bmsg0, Wupd0, bupd0,
           Wres1, Wmsg1, bmsg1, Wupd1, bupd1,
           Wp1, bp1, Wp2, bp2):
    row = edge_index[0]
    col = edge_index[1]
    ea = jnp.broadcast_to(edge_attr.reshape(E, 1), (E, L)).reshape(E * L)

    batch8 = jnp.broadcast_to(batch.reshape(1, N), (8, N))
    wp2pad = jnp.pad(Wp2, ((0, 0), (0, D - 1)))
    bp2b = jnp.broadcast_to(bp2.reshape(1, 1), (1, D))

    a, b = _tc_pre(x, Wmsg0[:D], Wmsg0[D:2 * D], bmsg0.reshape(1, D))
    parts0 = _sc_edge(a, b, Wmsg0[2 * D], col, row, ea)
    h1, a1, b1 = _tc_post_pre(
        x, parts0[0], parts0[1], Wres0, Wupd0[:D], Wupd0[D:],
        bupd0.reshape(1, D), Wmsg1[:D], Wmsg1[D:2 * D], bmsg1.reshape(1, D))
    parts1 = _sc_edge(a1, b1, Wmsg1[2 * D], col, row, ea)
    out = _tc_post_readout(
        h1, parts1[0], parts1[1], Wres1, Wupd1[:D], Wupd1[D:],
        bupd1.reshape(1, D), batch8, Wp1, bp1.reshape(1, D), wp2pad, bp2b)
    return out[:, :1]
